# baseline (speedup 1.0000x reference)
import jax
import jax.numpy as jnp
from jax.experimental import pallas as pl
from jax.experimental.pallas import tpu as pltpu

_BT = 256

_G1 = 1536
_G2 = 384


def _placement(n_in, n_pool):
    pos = jnp.arange(n_in)[:, None, None, None]
    par = jnp.arange(2)[None, :, None, None]
    ppos = jnp.arange(n_pool)[None, None, :, None]
    tap = jnp.arange(5)[None, None, None, :]
    return (pos == 2 * ppos + par + tap).astype(jnp.float32)


def _toeplitz1(w1):
    w = w1.reshape(10, 5, 5)
    p = _placement(28, 12)
    t = jnp.einsum('hapi,wcqj,oij->hwacopq', p, p, w)
    t = t.reshape(784, 4, 1440)
    t = jnp.pad(t, ((0, 0), (0, 0), (0, _G1 - 1440)))
    return t.reshape(784, 4 * _G1)


def _toeplitz2(w2):
    w = w2.reshape(20, 10, 5, 5)
    p = _placement(12, 4)
    t = jnp.einsum('yapi,xcqj,onij->nyxacopq', p, p, w)
    t = t.reshape(1440, 4, 320)
    t = jnp.pad(t, ((0, _G1 - 1440), (0, 0), (0, _G2 - 320)))
    return t.reshape(_G1, 4 * _G2)


def _net_kernel(x_ref, w1_ref, b1_ref, w2_ref, b2_ref,
                f1w_ref, f1b_ref, f2w_ref, f2b_ref,
                o_ref, y1_ref, a1_ref, y2_ref):
    f32 = jnp.float32
    y1_ref[...] = jnp.dot(x_ref[...], w1_ref[...], preferred_element_type=f32)
    p1 = jnp.maximum(
        jnp.maximum(y1_ref[:, 0 * _G1:1 * _G1], y1_ref[:, 1 * _G1:2 * _G1]),
        jnp.maximum(y1_ref[:, 2 * _G1:3 * _G1], y1_ref[:, 3 * _G1:4 * _G1]))
    a1_ref[...] = jnp.maximum(p1 + b1_ref[...], 0.0).astype(jnp.bfloat16)

    y2_ref[...] = jnp.dot(a1_ref[...], w2_ref[...], preferred_element_type=f32)
    p2 = jnp.maximum(
        jnp.maximum(y2_ref[:, 0 * _G2:1 * _G2], y2_ref[:, 1 * _G2:2 * _G2]),
        jnp.maximum(y2_ref[:, 2 * _G2:3 * _G2], y2_ref[:, 3 * _G2:4 * _G2]))
    a2 = jnp.maximum(p2 + b2_ref[...], 0.0)

    h = jnp.dot(a2, f1w_ref[...], preferred_element_type=f32)
    h = jnp.maximum(h + f1b_ref[...], 0.0)
    y = jnp.dot(h, f2w_ref[...], preferred_element_type=f32)
    y = y + f2b_ref[...]
    m = jnp.max(y, axis=1, keepdims=True)
    s = y - m
    lse = jnp.log(jnp.sum(jnp.exp(s), axis=1, keepdims=True))
    o_ref[...] = s - lse


def kernel(x_nchw, conv1_w, conv1_b, conv2_w, conv2_b,
           fc1_w, fc1_b, fc2_w, fc2_b):
    B = x_nchw.shape[0]
    Bp = ((B + _BT - 1) // _BT) * _BT
    x = x_nchw.astype(jnp.bfloat16).reshape(B, 784)
    if Bp != B:
        x = jnp.pad(x, ((0, Bp - B), (0, 0)))

    w1 = _toeplitz1(conv1_w.astype(jnp.float32)).astype(jnp.bfloat16)
    w2 = _toeplitz2(conv2_w.astype(jnp.float32)).astype(jnp.bfloat16)
    b1r = jnp.pad(jnp.repeat(conv1_b, 144), (0, _G1 - 1440)).reshape(1, _G1)
    b2r = jnp.pad(jnp.repeat(conv2_b, 16), (0, _G2 - 320)).reshape(1, _G2)
    f1w = jnp.pad(fc1_w.T, ((0, _G2 - 320), (0, 78)))
    f1b = jnp.pad(fc1_b.reshape(-1), (0, 78)).reshape(1, 128)
    f2w = jnp.pad(fc2_w.T, ((0, 78), (0, 0)))
    f2b = fc2_b.reshape(1, 10)

    out = pl.pallas_call(
        _net_kernel,
        out_shape=jax.ShapeDtypeStruct((Bp, 10), jnp.float32),
        grid=(Bp // _BT,),
        in_specs=[
            pl.BlockSpec((_BT, 784), lambda b: (b, 0)),
            pl.BlockSpec((784, 4 * _G1), lambda b: (0, 0)),
            pl.BlockSpec((1, _G1), lambda b: (0, 0)),
            pl.BlockSpec((_G1, 4 * _G2), lambda b: (0, 0)),
            pl.BlockSpec((1, _G2), lambda b: (0, 0)),
            pl.BlockSpec((_G2, 128), lambda b: (0, 0)),
            pl.BlockSpec((1, 128), lambda b: (0, 0)),
            pl.BlockSpec((128, 10), lambda b: (0, 0)),
            pl.BlockSpec((1, 10), lambda b: (0, 0)),
        ],
        out_specs=pl.BlockSpec((_BT, 10), lambda b: (b, 0)),
        scratch_shapes=[
            pltpu.VMEM((_BT, 4 * _G1), jnp.float32),
            pltpu.VMEM((_BT, _G1), jnp.bfloat16),
            pltpu.VMEM((_BT, 4 * _G2), jnp.float32),
        ],
        compiler_params=pltpu.CompilerParams(
            dimension_semantics=("parallel",)),
    )(x, w1, b1r, w2, b2r, f1w, f1b, f2w, f2b)
    return out[:B]


# speedup vs baseline: 1.0756x; 1.0756x over previous
import jax
import jax.numpy as jnp
from jax.experimental import pallas as pl
from jax.experimental.pallas import tpu as pltpu

_BT = 512

_G1 = 1536
_G2 = 384


def _placement(n_in, n_pool):
    pos = jnp.arange(n_in)[:, None, None, None]
    par = jnp.arange(2)[None, :, None, None]
    ppos = jnp.arange(n_pool)[None, None, :, None]
    tap = jnp.arange(5)[None, None, None, :]
    return (pos == 2 * ppos + par + tap).astype(jnp.float32)


def _toeplitz1(w1):
    w = w1.reshape(10, 5, 5)
    p = _placement(28, 12)
    t = jnp.einsum('hapi,wcqj,oij->hwacopq', p, p, w)
    t = t.reshape(784, 4, 1440)
    t = jnp.pad(t, ((0, 0), (0, 0), (0, _G1 - 1440)))
    return t.reshape(784, 4 * _G1)


def _toeplitz2(w2):
    w = w2.reshape(20, 10, 5, 5)
    p = _placement(12, 4)
    t = jnp.einsum('yapi,xcqj,onij->nyxacopq', p, p, w)
    t = t.reshape(1440, 4, 320)
    t = jnp.pad(t, ((0, _G1 - 1440), (0, 0), (0, _G2 - 320)))
    return t.reshape(_G1, 4 * _G2)


def _net_kernel(x_ref, w1_ref, b1_ref, w2_ref, b2_ref,
                f1w_ref, f1b_ref, f2w_ref, f2b_ref,
                o_ref, y1_ref, a1_ref, y2_ref):
    f32 = jnp.float32
    y1_ref[...] = jnp.dot(x_ref[...], w1_ref[...], preferred_element_type=f32)
    p1 = jnp.maximum(
        jnp.maximum(y1_ref[:, 0 * _G1:1 * _G1], y1_ref[:, 1 * _G1:2 * _G1]),
        jnp.maximum(y1_ref[:, 2 * _G1:3 * _G1], y1_ref[:, 3 * _G1:4 * _G1]))
    a1_ref[...] = jnp.maximum(p1 + b1_ref[...], 0.0)

    y2_ref[...] = jnp.dot(a1_ref[...], w2_ref[...], preferred_element_type=f32)
    p2 = jnp.maximum(
        jnp.maximum(y2_ref[:, 0 * _G2:1 * _G2], y2_ref[:, 1 * _G2:2 * _G2]),
        jnp.maximum(y2_ref[:, 2 * _G2:3 * _G2], y2_ref[:, 3 * _G2:4 * _G2]))
    a2 = jnp.maximum(p2 + b2_ref[...], 0.0)

    h = jnp.dot(a2, f1w_ref[...], preferred_element_type=f32)
    h = jnp.maximum(h + f1b_ref[...], 0.0)
    y = jnp.dot(h, f2w_ref[...], preferred_element_type=f32)
    y = y + f2b_ref[...]
    m = jnp.max(y, axis=1, keepdims=True)
    s = y - m
    lse = jnp.log(jnp.sum(jnp.exp(s), axis=1, keepdims=True))
    o_ref[...] = s - lse


def kernel(x_nchw, conv1_w, conv1_b, conv2_w, conv2_b,
           fc1_w, fc1_b, fc2_w, fc2_b):
    B = x_nchw.shape[0]
    Bp = ((B + _BT - 1) // _BT) * _BT
    x = x_nchw.astype(jnp.float32).reshape(B, 784)
    if Bp != B:
        x = jnp.pad(x, ((0, Bp - B), (0, 0)))

    w1 = _toeplitz1(conv1_w.astype(jnp.float32))
    w2 = _toeplitz2(conv2_w.astype(jnp.float32))
    b1r = jnp.pad(jnp.repeat(conv1_b, 144), (0, _G1 - 1440)).reshape(1, _G1)
    b2r = jnp.pad(jnp.repeat(conv2_b, 16), (0, _G2 - 320)).reshape(1, _G2)
    f1w = jnp.pad(fc1_w.T, ((0, _G2 - 320), (0, 78)))
    f1b = jnp.pad(fc1_b.reshape(-1), (0, 78)).reshape(1, 128)
    f2w = jnp.pad(fc2_w.T, ((0, 78), (0, 0)))
    f2b = fc2_b.reshape(1, 10)

    out = pl.pallas_call(
        _net_kernel,
        out_shape=jax.ShapeDtypeStruct((Bp, 10), jnp.float32),
        grid=(Bp // _BT,),
        in_specs=[
            pl.BlockSpec((_BT, 784), lambda b: (b, 0)),
            pl.BlockSpec((784, 4 * _G1), lambda b: (0, 0)),
            pl.BlockSpec((1, _G1), lambda b: (0, 0)),
            pl.BlockSpec((_G1, 4 * _G2), lambda b: (0, 0)),
            pl.BlockSpec((1, _G2), lambda b: (0, 0)),
            pl.BlockSpec((_G2, 128), lambda b: (0, 0)),
            pl.BlockSpec((1, 128), lambda b: (0, 0)),
            pl.BlockSpec((128, 10), lambda b: (0, 0)),
            pl.BlockSpec((1, 10), lambda b: (0, 0)),
        ],
        out_specs=pl.BlockSpec((_BT, 10), lambda b: (b, 0)),
        scratch_shapes=[
            pltpu.VMEM((_BT, 4 * _G1), jnp.float32),
            pltpu.VMEM((_BT, _G1), jnp.float32),
            pltpu.VMEM((_BT, 4 * _G2), jnp.float32),
        ],
        compiler_params=pltpu.CompilerParams(
            dimension_semantics=("parallel",)),
    )(x, w1, b1r, w2, b2r, f1w, f1b, f2w, f2b)
    return out[:B]
